# baseline (device time: 49778 ns/iter reference)
import jax
import jax.numpy as jnp
from jax import lax
from jax.experimental import pallas as pl
from jax.experimental.pallas import tpu as pltpu

N_DEV = 4
B = 2
SQ = 128
SKV_PER = 128
HQ = 4
DH = 64
WINDOW = 128
D_MODEL = 512
D_QK = 256
ROWS = B * HQ * SKV_PER


def kernel(x, Wq, K_ext, V_ext, Wo):
    kt = jnp.transpose(K_ext, (0, 2, 1, 3)).reshape(ROWS, DH)
    vt = jnp.transpose(V_ext, (0, 2, 1, 3)).reshape(ROWS, DH)
    kv = jnp.concatenate([kt, vt], axis=0)

    def body(x_ref, wq_ref, kv_ref, wo_ref, out_ref, gath_ref,
             send_sems, recv_sems):
        my_pos = lax.axis_index("i")
        left = lax.rem(my_pos + N_DEV - 1, N_DEV)
        right = lax.rem(my_pos + 1, N_DEV)

        barrier_sem = pltpu.get_barrier_semaphore()
        pl.semaphore_signal(barrier_sem, inc=1, device_id=(left,),
                            device_id_type=pl.DeviceIdType.MESH)
        pl.semaphore_signal(barrier_sem, inc=1, device_id=(right,),
                            device_id_type=pl.DeviceIdType.MESH)
        pl.semaphore_wait(barrier_sem, 2)

        gath_ref[0] = kv_ref[...]

        for t in range(N_DEV - 1):
            rdma = pltpu.make_async_remote_copy(
                src_ref=gath_ref.at[t],
                dst_ref=gath_ref.at[t + 1],
                send_sem=send_sems.at[t],
                recv_sem=recv_sems.at[t],
                device_id=(right,),
                device_id_type=pl.DeviceIdType.MESH,
            )
            rdma.start()
            rdma.wait()

        qi = lax.broadcasted_iota(jnp.int32, (SQ, SKV_PER), 0)
        kj = lax.broadcasted_iota(jnp.int32, (SQ, SKV_PER), 1)
        for b in range(B):
            q_b = jnp.dot(x_ref[b], wq_ref[...],
                          preferred_element_type=jnp.float32)
            ctx_parts = []
            for h in range(HQ):
                q_bh = q_b[:, h * DH:(h + 1) * DH]
                r0 = (b * HQ + h) * SKV_PER
                score_blocks = []
                v_blocks = []
                for r in range(N_DEV):
                    origin = lax.rem(my_pos - r + N_DEV, N_DEV)
                    k_blk = gath_ref[r, r0:r0 + SKV_PER, :]
                    v_blk = gath_ref[r, ROWS + r0:ROWS + r0 + SKV_PER, :]
                    s_blk = jnp.dot(q_bh, k_blk.T,
                                    preferred_element_type=jnp.float32)
                    s_blk = s_blk * 0.125
                    kg = kj + origin * SKV_PER
                    s_blk = jnp.where(jnp.abs(qi - kg) <= WINDOW,
                                      s_blk, -1e9)
                    score_blocks.append(s_blk)
                    v_blocks.append(v_blk)
                scores = jnp.concatenate(score_blocks, axis=1)
                vfull = jnp.concatenate(v_blocks, axis=0)
                smax = jnp.max(scores, axis=1, keepdims=True)
                w = jnp.exp(scores - smax)
                w = w / jnp.sum(w, axis=1, keepdims=True)
                ctx_parts.append(jnp.dot(w, vfull,
                                         preferred_element_type=jnp.float32))
            ctx_b = jnp.concatenate(ctx_parts, axis=1)
            out_ref[b] = jnp.dot(ctx_b, wo_ref[...],
                                 preferred_element_type=jnp.float32)

    return pl.pallas_call(
        body,
        out_shape=jax.ShapeDtypeStruct((B, SQ, D_MODEL), jnp.float32),
        in_specs=[pl.BlockSpec(memory_space=pltpu.VMEM)] * 4,
        out_specs=pl.BlockSpec(memory_space=pltpu.VMEM),
        scratch_shapes=[
            pltpu.VMEM((N_DEV, 2 * ROWS, DH), jnp.float32),
            pltpu.SemaphoreType.DMA((N_DEV - 1,)),
            pltpu.SemaphoreType.DMA((N_DEV - 1,)),
        ],
        compiler_params=pltpu.CompilerParams(collective_id=0),
    )(x, Wq, kv, Wo)


# device time: 33127 ns/iter; 1.5026x vs baseline; 1.5026x over previous
import jax
import jax.numpy as jnp
from jax import lax
from jax.experimental import pallas as pl
from jax.experimental.pallas import tpu as pltpu

N_DEV = 4
B = 2
SQ = 128
SKV_PER = 128
HQ = 4
DH = 64
WINDOW = 128
D_MODEL = 512
D_QK = 256
ROWS = B * HQ * SKV_PER

SF = 0
SS = 1


def kernel(x, Wq, K_ext, V_ext, Wo):
    kt = jnp.transpose(K_ext, (0, 2, 1, 3)).reshape(ROWS, DH)
    vt = jnp.transpose(V_ext, (0, 2, 1, 3)).reshape(ROWS, DH)
    kv = jnp.concatenate([kt, vt], axis=0)

    def body(x_ref, wq_ref, kv_ref, wo_ref, out_ref, gath_ref,
             send_sems, recv_sems):
        my_pos = lax.axis_index("i")
        left = lax.rem(my_pos + N_DEV - 1, N_DEV)
        right = lax.rem(my_pos + 1, N_DEV)

        barrier_sem = pltpu.get_barrier_semaphore()
        pl.semaphore_signal(barrier_sem, inc=1, device_id=(left,),
                            device_id_type=pl.DeviceIdType.MESH)
        pl.semaphore_signal(barrier_sem, inc=1, device_id=(right,),
                            device_id_type=pl.DeviceIdType.MESH)
        pl.semaphore_wait(barrier_sem, 2)

        def start_send(dst_slot, send_idx, dev):
            pltpu.make_async_remote_copy(
                src_ref=gath_ref.at[SF],
                dst_ref=gath_ref.at[dst_slot],
                send_sem=send_sems.at[send_idx],
                recv_sem=recv_sems.at[dst_slot],
                device_id=(dev,),
                device_id_type=pl.DeviceIdType.MESH,
            ).start()

        def wait_recv(slot):
            pltpu.make_async_remote_copy(
                src_ref=gath_ref.at[slot], dst_ref=gath_ref.at[slot],
                send_sem=send_sems.at[0], recv_sem=recv_sems.at[slot],
                device_id=(my_pos,), device_id_type=pl.DeviceIdType.MESH,
            ).wait_recv()

        def wait_send(send_idx):
            pltpu.make_async_remote_copy(
                src_ref=gath_ref.at[SF], dst_ref=gath_ref.at[SF],
                send_sem=send_sems.at[send_idx], recv_sem=recv_sems.at[SF],
                device_id=(my_pos,), device_id_type=pl.DeviceIdType.MESH,
            ).wait_send()

        @pl.when(my_pos == 0)
        def _():
            gath_ref[SF] = kv_ref[...]
            start_send(SS, 0, 1)
            start_send(SF, 1, 3)

        @pl.when(my_pos == 1)
        def _():
            gath_ref[SF] = kv_ref[...]
            start_send(SS, 0, 0)
            start_send(SF, 1, 2)

        q_proj = [
            jnp.dot(x_ref[b], wq_ref[...],
                    preferred_element_type=jnp.float32)
            for b in range(B)
        ]

        @pl.when(my_pos == 2)
        def _():
            wait_recv(SF)
            start_send(SS, 0, 3)

        @pl.when(my_pos == 3)
        def _():
            wait_recv(SF)
            start_send(SS, 0, 2)

        f_is_1 = (my_pos == 1) | (my_pos == 2)
        f_off = jnp.where(f_is_1, SKV_PER, 0)
        s_off = jnp.where(f_is_1, 0, SKV_PER)

        qi = lax.broadcasted_iota(jnp.int32, (SQ, SKV_PER), 0)
        kj = lax.broadcasted_iota(jnp.int32, (SQ, SKV_PER), 1)

        def block_scores(slot, off, b, h):
            r0 = (b * HQ + h) * SKV_PER
            k_blk = gath_ref[slot, r0:r0 + SKV_PER, :]
            v_blk = gath_ref[slot, ROWS + r0:ROWS + r0 + SKV_PER, :]
            q_bh = q_proj[b][:, h * DH:(h + 1) * DH]
            s = jnp.dot(q_bh, k_blk.T,
                        preferred_element_type=jnp.float32) * 0.125
            s = jnp.where(jnp.abs(qi - (kj + off)) <= WINDOW, s, -1e9)
            return s, v_blk

        parts = []
        for b in range(B):
            for h in range(HQ):
                s_f, v_f = block_scores(SF, f_off, b, h)
                m_f = jnp.max(s_f, axis=1, keepdims=True)
                p_f = jnp.exp(s_f - m_f)
                l_f = jnp.sum(p_f, axis=1, keepdims=True)
                c_f = jnp.dot(p_f, v_f, preferred_element_type=jnp.float32)
                parts.append((m_f, l_f, c_f))

        wait_recv(SS)
        for b in range(B):
            ctx_heads = []
            for h in range(HQ):
                m_f, l_f, c_f = parts[b * HQ + h]
                s_s, v_s = block_scores(SS, s_off, b, h)
                m = jnp.maximum(m_f, jnp.max(s_s, axis=1, keepdims=True))
                alpha = jnp.exp(m_f - m)
                p_s = jnp.exp(s_s - m)
                c = c_f * alpha + jnp.dot(p_s, v_s,
                                          preferred_element_type=jnp.float32)
                l = l_f * alpha + jnp.sum(p_s, axis=1, keepdims=True)
                ctx_heads.append(c / l)
            ctx_b = jnp.concatenate(ctx_heads, axis=1)
            out_ref[b] = jnp.dot(ctx_b, wo_ref[...],
                                 preferred_element_type=jnp.float32)

        @pl.when((my_pos == 0) | (my_pos == 1))
        def _():
            wait_send(0)
            wait_send(1)

        @pl.when((my_pos == 2) | (my_pos == 3))
        def _():
            wait_send(0)

    return pl.pallas_call(
        body,
        out_shape=jax.ShapeDtypeStruct((B, SQ, D_MODEL), jnp.float32),
        in_specs=[pl.BlockSpec(memory_space=pltpu.VMEM)] * 4,
        out_specs=pl.BlockSpec(memory_space=pltpu.VMEM),
        scratch_shapes=[
            pltpu.VMEM((2, 2 * ROWS, DH), jnp.float32),
            pltpu.SemaphoreType.DMA((2,)),
            pltpu.SemaphoreType.DMA((2,)),
        ],
        compiler_params=pltpu.CompilerParams(collective_id=0),
    )(x, Wq, kv, Wo)


# device time: 30810 ns/iter; 1.6156x vs baseline; 1.0752x over previous
import os

import jax
import jax.numpy as jnp
from jax import lax
from jax.experimental import pallas as pl
from jax.experimental.pallas import tpu as pltpu

_MODE = os.environ.get("KMODE", "full")

N_DEV = 4
B = 2
SQ = 128
SKV_PER = 128
HQ = 4
DH = 64
WINDOW = 128
D_MODEL = 512
D_QK = 256
ROWS = B * HQ * SKV_PER

SF = 0
SS = 1


def kernel(x, Wq, K_ext, V_ext, Wo):
    kt = jnp.transpose(K_ext, (0, 2, 1, 3)).reshape(ROWS, DH)
    vt = jnp.transpose(V_ext, (0, 2, 1, 3)).reshape(ROWS, DH)
    kv = jnp.concatenate([kt, vt], axis=0)

    def body(x_ref, wq_ref, kv_ref, wo_ref, out_ref, gath_ref,
             send_sems, recv_sems):
        my_pos = lax.axis_index("i")
        left = lax.rem(my_pos + N_DEV - 1, N_DEV)
        right = lax.rem(my_pos + 1, N_DEV)

        if _MODE != "nocomm":
            barrier_sem = pltpu.get_barrier_semaphore()
            pl.semaphore_signal(barrier_sem, inc=1, device_id=(left,),
                                device_id_type=pl.DeviceIdType.MESH)
            pl.semaphore_signal(barrier_sem, inc=1, device_id=(right,),
                                device_id_type=pl.DeviceIdType.MESH)
            pl.semaphore_wait(barrier_sem, 2)

        def start_send(dst_slot, send_idx, dev):
            pltpu.make_async_remote_copy(
                src_ref=gath_ref.at[SF],
                dst_ref=gath_ref.at[dst_slot],
                send_sem=send_sems.at[send_idx],
                recv_sem=recv_sems.at[dst_slot],
                device_id=(dev,),
                device_id_type=pl.DeviceIdType.MESH,
            ).start()

        def wait_recv(slot):
            pltpu.make_async_remote_copy(
                src_ref=gath_ref.at[slot], dst_ref=gath_ref.at[slot],
                send_sem=send_sems.at[0], recv_sem=recv_sems.at[slot],
                device_id=(my_pos,), device_id_type=pl.DeviceIdType.MESH,
            ).wait_recv()

        def wait_send(send_idx):
            pltpu.make_async_remote_copy(
                src_ref=gath_ref.at[SF], dst_ref=gath_ref.at[SF],
                send_sem=send_sems.at[send_idx], recv_sem=recv_sems.at[SF],
                device_id=(my_pos,), device_id_type=pl.DeviceIdType.MESH,
            ).wait_send()

        if _MODE == "nocomm":
            gath_ref[SF] = kv_ref[...]
            gath_ref[SS] = kv_ref[...]
        else:
            @pl.when(my_pos == 0)
            def _():
                gath_ref[SF] = kv_ref[...]
                start_send(SS, 0, 1)
                start_send(SF, 1, 3)

            @pl.when(my_pos == 1)
            def _():
                gath_ref[SF] = kv_ref[...]
                start_send(SS, 0, 0)
                start_send(SF, 1, 2)

        q_proj = [
            jnp.dot(x_ref[b], wq_ref[...],
                    preferred_element_type=jnp.float32)
            for b in range(B)
        ]

        if _MODE != "nocomm":
            @pl.when(my_pos == 2)
            def _():
                wait_recv(SF)
                start_send(SS, 0, 3)

            @pl.when(my_pos == 3)
            def _():
                wait_recv(SF)
                start_send(SS, 0, 2)

        f_is_1 = (my_pos == 1) | (my_pos == 2)
        f_off = jnp.where(f_is_1, SKV_PER, 0)
        s_off = jnp.where(f_is_1, 0, SKV_PER)

        qi = lax.broadcasted_iota(jnp.int32, (SQ, SKV_PER), 0)
        kj = lax.broadcasted_iota(jnp.int32, (SQ, SKV_PER), 1)

        def block_scores(slot, off, b, h):
            r0 = (b * HQ + h) * SKV_PER
            k_blk = gath_ref[slot, r0:r0 + SKV_PER, :]
            v_blk = gath_ref[slot, ROWS + r0:ROWS + r0 + SKV_PER, :]
            q_bh = q_proj[b][:, h * DH:(h + 1) * DH]
            s = jnp.dot(q_bh, k_blk.T,
                        preferred_element_type=jnp.float32) * 0.125
            s = jnp.where(jnp.abs(qi - (kj + off)) <= WINDOW, s, -1e9)
            return s, v_blk

        if _MODE != "nocompute":
            parts = []
            for b in range(B):
                for h in range(HQ):
                    s_f, v_f = block_scores(SF, f_off, b, h)
                    m_f = jnp.max(s_f, axis=1, keepdims=True)
                    p_f = jnp.exp(s_f - m_f)
                    l_f = jnp.sum(p_f, axis=1, keepdims=True)
                    c_f = jnp.dot(p_f, v_f,
                                  preferred_element_type=jnp.float32)
                    parts.append((m_f, l_f, c_f))

        if _MODE != "nocomm":
            wait_recv(SS)

        if _MODE == "nocompute":
            out_ref[...] = jnp.zeros((B, SQ, D_MODEL), jnp.float32)
        else:
            for b in range(B):
                ctx_heads = []
                for h in range(HQ):
                    m_f, l_f, c_f = parts[b * HQ + h]
                    s_s, v_s = block_scores(SS, s_off, b, h)
                    m = jnp.maximum(m_f,
                                    jnp.max(s_s, axis=1, keepdims=True))
                    alpha = jnp.exp(m_f - m)
                    p_s = jnp.exp(s_s - m)
                    c = c_f * alpha + jnp.dot(
                        p_s, v_s, preferred_element_type=jnp.float32)
                    l = l_f * alpha + jnp.sum(p_s, axis=1, keepdims=True)
                    ctx_heads.append(c / l)
                ctx_b = jnp.concatenate(ctx_heads, axis=1)
                out_ref[b] = jnp.dot(ctx_b, wo_ref[...],
                                     preferred_element_type=jnp.float32)

        if _MODE != "nocomm":
            @pl.when((my_pos == 0) | (my_pos == 1))
            def _():
                wait_send(0)
                wait_send(1)

            @pl.when((my_pos == 2) | (my_pos == 3))
            def _():
                wait_send(0)

    return pl.pallas_call(
        body,
        out_shape=jax.ShapeDtypeStruct((B, SQ, D_MODEL), jnp.float32),
        in_specs=[pl.BlockSpec(memory_space=pltpu.VMEM)] * 4,
        out_specs=pl.BlockSpec(memory_space=pltpu.VMEM),
        scratch_shapes=[
            pltpu.VMEM((2, 2 * ROWS, DH), jnp.float32),
            pltpu.SemaphoreType.DMA((2,)),
            pltpu.SemaphoreType.DMA((2,)),
        ],
        compiler_params=pltpu.CompilerParams(collective_id=0),
    )(x, Wq, kv, Wo)


# device time: 20336 ns/iter; 2.4478x vs baseline; 1.5150x over previous
import os

import jax
import jax.numpy as jnp
from jax import lax
from jax.experimental import pallas as pl
from jax.experimental.pallas import tpu as pltpu

_MODE = os.environ.get("KMODE", "full")

N_DEV = 4
B = 2
SQ = 128
SKV_PER = 128
HQ = 4
DH = 64
WINDOW = 128
D_MODEL = 512
D_QK = 256
ROWS_T = B * HQ * DH

SF = 0
SS = 1


def kernel(x, Wq, K_ext, V_ext, Wo):
    kT = jnp.transpose(K_ext, (0, 2, 3, 1)).reshape(ROWS_T, SKV_PER)
    vT = jnp.transpose(V_ext, (0, 2, 3, 1)).reshape(ROWS_T, SKV_PER)
    kv = jnp.concatenate([kT, vT], axis=0)

    def body(x_ref, wq_ref, kv_ref, wo_ref, out_ref, gath_ref,
             send_sems, recv_sems):
        my_pos = lax.axis_index("i")
        left = lax.rem(my_pos + N_DEV - 1, N_DEV)
        right = lax.rem(my_pos + 1, N_DEV)

        if _MODE != "nocomm":
            barrier_sem = pltpu.get_barrier_semaphore()
            pl.semaphore_signal(barrier_sem, inc=1, device_id=(left,),
                                device_id_type=pl.DeviceIdType.MESH)
            pl.semaphore_signal(barrier_sem, inc=1, device_id=(right,),
                                device_id_type=pl.DeviceIdType.MESH)
            pl.semaphore_wait(barrier_sem, 2)

        def start_send(dst_slot, send_idx, dev):
            pltpu.make_async_remote_copy(
                src_ref=gath_ref.at[SF],
                dst_ref=gath_ref.at[dst_slot],
                send_sem=send_sems.at[send_idx],
                recv_sem=recv_sems.at[dst_slot],
                device_id=(dev,),
                device_id_type=pl.DeviceIdType.MESH,
            ).start()

        def wait_recv(slot):
            pltpu.make_async_remote_copy(
                src_ref=gath_ref.at[slot], dst_ref=gath_ref.at[slot],
                send_sem=send_sems.at[0], recv_sem=recv_sems.at[slot],
                device_id=(my_pos,), device_id_type=pl.DeviceIdType.MESH,
            ).wait_recv()

        def wait_send(send_idx):
            pltpu.make_async_remote_copy(
                src_ref=gath_ref.at[SF], dst_ref=gath_ref.at[SF],
                send_sem=send_sems.at[send_idx], recv_sem=recv_sems.at[SF],
                device_id=(my_pos,), device_id_type=pl.DeviceIdType.MESH,
            ).wait_send()

        if _MODE == "nocomm":
            gath_ref[SF] = kv_ref[...]
            gath_ref[SS] = kv_ref[...]
        else:
            @pl.when(my_pos == 0)
            def _():
                gath_ref[SF] = kv_ref[...]
                start_send(SF, 1, 3)
                start_send(SS, 0, 1)

            @pl.when(my_pos == 1)
            def _():
                gath_ref[SF] = kv_ref[...]
                start_send(SF, 1, 2)
                start_send(SS, 0, 0)

        q_proj = [
            jnp.dot(x_ref[b], wq_ref[...],
                    preferred_element_type=jnp.float32)
            for b in range(B)
        ]

        if _MODE != "nocomm":
            @pl.when(my_pos == 2)
            def _():
                wait_recv(SF)
                start_send(SS, 0, 3)

            @pl.when(my_pos == 3)
            def _():
                wait_recv(SF)
                start_send(SS, 0, 2)

        f_is_1 = (my_pos == 1) | (my_pos == 2)
        f_off = jnp.where(f_is_1, SKV_PER, 0)
        s_off = jnp.where(f_is_1, 0, SKV_PER)

        qi = lax.broadcasted_iota(jnp.int32, (SQ, SKV_PER), 0)
        kj = lax.broadcasted_iota(jnp.int32, (SQ, SKV_PER), 1)

        def block_scores(slot, off, b, h):
            r0 = (b * HQ + h) * DH
            kT_blk = gath_ref[slot, r0:r0 + DH, :]
            vT_blk = gath_ref[slot, ROWS_T + r0:ROWS_T + r0 + DH, :]
            q_bh = q_proj[b][:, h * DH:(h + 1) * DH]
            s = jnp.dot(q_bh, kT_blk,
                        preferred_element_type=jnp.float32) * 0.125
            s = jnp.where(jnp.abs(qi - (kj + off)) <= WINDOW, s, -1e9)
            return s, vT_blk

        def pv(p, vT_blk):
            return lax.dot_general(
                p, vT_blk, (((1,), (1,)), ((), ())),
                preferred_element_type=jnp.float32)

        if _MODE != "nocompute":
            parts = []
            for b in range(B):
                for h in range(HQ):
                    s_f, vT_f = block_scores(SF, f_off, b, h)
                    m_f = jnp.max(s_f, axis=1, keepdims=True)
                    p_f = jnp.exp(s_f - m_f)
                    l_f = jnp.sum(p_f, axis=1, keepdims=True)
                    parts.append((m_f, l_f, pv(p_f, vT_f)))

        if _MODE != "nocomm":
            wait_recv(SS)

        if _MODE == "nocompute":
            out_ref[...] = jnp.zeros((B, SQ, D_MODEL), jnp.float32)
        else:
            for b in range(B):
                ctx_heads = []
                for h in range(HQ):
                    m_f, l_f, c_f = parts[b * HQ + h]
                    s_s, vT_s = block_scores(SS, s_off, b, h)
                    m = jnp.maximum(m_f,
                                    jnp.max(s_s, axis=1, keepdims=True))
                    alpha = jnp.exp(m_f - m)
                    p_s = jnp.exp(s_s - m)
                    c = c_f * alpha + pv(p_s, vT_s)
                    l = l_f * alpha + jnp.sum(p_s, axis=1, keepdims=True)
                    ctx_heads.append(c / l)
                ctx_b = jnp.concatenate(ctx_heads, axis=1)
                out_ref[b] = jnp.dot(ctx_b, wo_ref[...],
                                     preferred_element_type=jnp.float32)

        if _MODE != "nocomm":
            @pl.when((my_pos == 0) | (my_pos == 1))
            def _():
                wait_send(0)
                wait_send(1)

            @pl.when((my_pos == 2) | (my_pos == 3))
            def _():
                wait_send(0)

    return pl.pallas_call(
        body,
        out_shape=jax.ShapeDtypeStruct((B, SQ, D_MODEL), jnp.float32),
        in_specs=[pl.BlockSpec(memory_space=pltpu.VMEM)] * 4,
        out_specs=pl.BlockSpec(memory_space=pltpu.VMEM),
        scratch_shapes=[
            pltpu.VMEM((2, 2 * ROWS_T, SKV_PER), jnp.float32),
            pltpu.SemaphoreType.DMA((2,)),
            pltpu.SemaphoreType.DMA((2,)),
        ],
        compiler_params=pltpu.CompilerParams(collective_id=0),
    )(x, Wq, kv, Wo)


# device time: 15698 ns/iter; 3.1710x vs baseline; 1.2955x over previous
import os

import jax
import jax.numpy as jnp
from jax import lax
from jax.experimental import pallas as pl
from jax.experimental.pallas import tpu as pltpu

_MODE = os.environ.get("KMODE", "full")

N_DEV = 4
B = 2
SQ = 128
SKV_PER = 128
HQ = 4
DH = 64
WINDOW = 128
D_MODEL = 512
D_QK = 256
ROWS_T = B * HQ * DH

SF = 0
SS = 1


def kernel(x, Wq, K_ext, V_ext, Wo):
    kT = jnp.transpose(K_ext, (0, 2, 3, 1)).reshape(ROWS_T, SKV_PER)
    vT = jnp.transpose(V_ext, (0, 2, 3, 1)).reshape(ROWS_T, SKV_PER)
    kv = jnp.concatenate([kT, vT], axis=0).astype(jnp.bfloat16)
    x = x.astype(jnp.bfloat16)
    Wq = Wq.astype(jnp.bfloat16)
    Wo = Wo.astype(jnp.bfloat16)

    def body(x_ref, wq_ref, kv_ref, wo_ref, out_ref, gath_ref,
             send_sems, recv_sems):
        my_pos = lax.axis_index("i")
        left = lax.rem(my_pos + N_DEV - 1, N_DEV)
        right = lax.rem(my_pos + 1, N_DEV)

        if _MODE != "nocomm":
            barrier_sem = pltpu.get_barrier_semaphore()
            pl.semaphore_signal(barrier_sem, inc=1, device_id=(left,),
                                device_id_type=pl.DeviceIdType.MESH)
            pl.semaphore_signal(barrier_sem, inc=1, device_id=(right,),
                                device_id_type=pl.DeviceIdType.MESH)
            pl.semaphore_wait(barrier_sem, 2)

        def start_send(dst_slot, send_idx, dev):
            pltpu.make_async_remote_copy(
                src_ref=gath_ref.at[SF],
                dst_ref=gath_ref.at[dst_slot],
                send_sem=send_sems.at[send_idx],
                recv_sem=recv_sems.at[dst_slot],
                device_id=(dev,),
                device_id_type=pl.DeviceIdType.MESH,
            ).start()

        def wait_recv(slot):
            pltpu.make_async_remote_copy(
                src_ref=gath_ref.at[slot], dst_ref=gath_ref.at[slot],
                send_sem=send_sems.at[0], recv_sem=recv_sems.at[slot],
                device_id=(my_pos,), device_id_type=pl.DeviceIdType.MESH,
            ).wait_recv()

        def wait_send(send_idx):
            pltpu.make_async_remote_copy(
                src_ref=gath_ref.at[SF], dst_ref=gath_ref.at[SF],
                send_sem=send_sems.at[send_idx], recv_sem=recv_sems.at[SF],
                device_id=(my_pos,), device_id_type=pl.DeviceIdType.MESH,
            ).wait_send()

        if _MODE == "nocomm":
            gath_ref[SF] = kv_ref[...]
            gath_ref[SS] = kv_ref[...]
        else:
            @pl.when(my_pos == 0)
            def _():
                gath_ref[SF] = kv_ref[...]
                start_send(SF, 1, 3)
                start_send(SS, 0, 1)

            @pl.when(my_pos == 1)
            def _():
                gath_ref[SF] = kv_ref[...]
                start_send(SF, 1, 2)
                start_send(SS, 0, 0)

        q_proj = [
            jnp.dot(x_ref[b], wq_ref[...],
                    preferred_element_type=jnp.float32
                    ).astype(jnp.bfloat16)
            for b in range(B)
        ]

        if _MODE != "nocomm":
            @pl.when(my_pos == 2)
            def _():
                wait_recv(SF)
                start_send(SS, 0, 3)

            @pl.when(my_pos == 3)
            def _():
                wait_recv(SF)
                start_send(SS, 0, 2)

        f_is_1 = (my_pos == 1) | (my_pos == 2)
        f_off = jnp.where(f_is_1, SKV_PER, 0)
        s_off = jnp.where(f_is_1, 0, SKV_PER)

        qi = lax.broadcasted_iota(jnp.int32, (SQ, SKV_PER), 0)
        kj = lax.broadcasted_iota(jnp.int32, (SQ, SKV_PER), 1)

        def block_scores(slot, off, b, h):
            r0 = (b * HQ + h) * DH
            kT_blk = gath_ref[slot, r0:r0 + DH, :]
            vT_blk = gath_ref[slot, ROWS_T + r0:ROWS_T + r0 + DH, :]
            q_bh = q_proj[b][:, h * DH:(h + 1) * DH]
            s = jnp.dot(q_bh, kT_blk,
                        preferred_element_type=jnp.float32) * 0.125
            s = jnp.where(jnp.abs(qi - (kj + off)) <= WINDOW, s, -1e9)
            return s, vT_blk

        def pv(p, vT_blk):
            return lax.dot_general(
                p.astype(jnp.bfloat16), vT_blk, (((1,), (1,)), ((), ())),
                preferred_element_type=jnp.float32)

        if _MODE != "nocompute":
            parts = []
            for b in range(B):
                for h in range(HQ):
                    s_f, vT_f = block_scores(SF, f_off, b, h)
                    m_f = jnp.max(s_f, axis=1, keepdims=True)
                    p_f = jnp.exp(s_f - m_f)
                    l_f = jnp.sum(p_f, axis=1, keepdims=True)
                    parts.append((m_f, l_f, pv(p_f, vT_f)))

        if _MODE != "nocomm":
            wait_recv(SS)

        if _MODE == "nocompute":
            out_ref[...] = jnp.zeros((B, SQ, D_MODEL), jnp.float32)
        else:
            for b in range(B):
                ctx_heads = []
                for h in range(HQ):
                    m_f, l_f, c_f = parts[b * HQ + h]
                    s_s, vT_s = block_scores(SS, s_off, b, h)
                    m = jnp.maximum(m_f,
                                    jnp.max(s_s, axis=1, keepdims=True))
                    alpha = jnp.exp(m_f - m)
                    p_s = jnp.exp(s_s - m)
                    c = c_f * alpha + pv(p_s, vT_s)
                    l = l_f * alpha + jnp.sum(p_s, axis=1, keepdims=True)
                    ctx_heads.append(c / l)
                ctx_b = jnp.concatenate(ctx_heads, axis=1)
                out_ref[b] = jnp.dot(ctx_b.astype(jnp.bfloat16),
                                     wo_ref[...],
                                     preferred_element_type=jnp.float32)

        if _MODE != "nocomm":
            @pl.when((my_pos == 0) | (my_pos == 1))
            def _():
                wait_send(0)
                wait_send(1)

            @pl.when((my_pos == 2) | (my_pos == 3))
            def _():
                wait_send(0)

    return pl.pallas_call(
        body,
        out_shape=jax.ShapeDtypeStruct((B, SQ, D_MODEL), jnp.float32),
        in_specs=[pl.BlockSpec(memory_space=pltpu.VMEM)] * 4,
        out_specs=pl.BlockSpec(memory_space=pltpu.VMEM),
        scratch_shapes=[
            pltpu.VMEM((2, 2 * ROWS_T, SKV_PER), jnp.bfloat16),
            pltpu.SemaphoreType.DMA((2,)),
            pltpu.SemaphoreType.DMA((2,)),
        ],
        compiler_params=pltpu.CompilerParams(collective_id=0),
    )(x, Wq, kv, Wo)
